# baseline (speedup 1.0000x reference)
import functools

import jax
import jax.numpy as jnp
from jax import lax
from jax.experimental import pallas as pl
from jax.experimental.pallas import tpu as pltpu
from jax.experimental.pallas import tpu_sc as plsc

VOCAB = 1000000
EMBED = 64
NUM_CLASSES = 128
B = 4096
T = 819200
AVG_DOC_LEN = 200.0
NUM_DOCS = 1000000
K1 = 1.2
BB = 0.75

NC = 2
NS = 16
NW = NC * NS
CHUNK = 128
BLK = 512
NCH = BLK // CHUNK
TOK_PER_W = T // NW
NBLK = TOK_PER_W // BLK
ROWS2D = T // CHUNK
ROWS_PER_W = ROWS2D // NW
DOCS_PER_S = B // NS

ABLK = 1024
ANCH = ABLK // CHUNK
NABLK = TOK_PER_W // ABLK

VPAD = 1048576


def _idf_body(df_ref, out_ref):
  x = df_ref[...]
  out_ref[...] = jnp.log((NUM_DOCS - x + 0.5) / (x + 0.5) + 1.0)


TCHUNK = 8192


def _tr_body(in_ref, out_ref):
  x = in_ref[...]
  eye = (lax.broadcasted_iota(jnp.int32, (EMBED, EMBED), 0) ==
         lax.broadcasted_iota(jnp.int32, (EMBED, EMBED), 1)
         ).astype(jnp.float32)
  y = lax.dot_general(x, eye, (((0,), (0,)), ((), ())),
                      preferred_element_type=jnp.float32)
  out_ref[:, 0:EMBED] = y[0:TCHUNK // 2, :]
  out_ref[:, EMBED:2 * EMBED] = y[TCHUNK // 2:TCHUNK, :]


_GATHER_DNUMS = lax.GatherDimensionNumbers(
    offset_dims=(), collapsed_slice_dims=(0,), start_index_map=(0,))


def _bcast_lane(v, j):
  idx = jnp.full((16, 1), j, dtype=jnp.int32)
  return lax.gather(v, idx, _GATHER_DNUMS, (1,),
                    mode=lax.GatherScatterMode.PROMISE_IN_BOUNDS)


def _sc_a_body(idx_hbm, tf_hbm, dl_hbm, df_hbm, w_hbm, gidx_hbm,
               idx_v, tf_v, dl_v, df_v, w_v, gidx_v, sem_meta, sem_gather,
               sem_out):
  c = lax.axis_index("c")
  s = lax.axis_index("s")
  wid = s * NC + c
  row_base = wid * (ROWS_PER_W)

  def meta_descs(g):
    m = g % 2
    sl = pl.ds(row_base + g * ANCH, ANCH)
    return [
        (idx_hbm.at[sl], idx_v.at[m]),
        (tf_hbm.at[sl], tf_v.at[m]),
        (dl_hbm.at[sl], dl_v.at[m]),
    ]

  def issue_meta(g):
    for src, dst in meta_descs(g):
      pltpu.async_copy(src, dst, sem_meta)

  def wait_meta(g):
    for src, dst in meta_descs(g):
      pltpu.make_async_copy(src, dst, sem_meta).wait()

  def gather_descs(g):
    m = g % 2
    return [(df_hbm.at[idx_v.at[m].at[k]], df_v.at[k]) for k in range(ANCH)]

  def out_descs(g):
    m = g % 2
    sl = pl.ds(row_base + g * ANCH, ANCH)
    return [(w_v.at[m], w_hbm.at[sl]), (gidx_v.at[m], gidx_hbm.at[sl])]

  issue_meta(0)

  def block(g, _):
    m = g % 2
    wait_meta(g)

    @pl.when(g + 1 < NABLK)
    def _():
      issue_meta(g + 1)

    for src, dst in gather_descs(g):
      pltpu.async_copy(src, dst, sem_gather)
    for src, dst in gather_descs(g):
      pltpu.make_async_copy(src, dst, sem_gather).wait()

    @pl.when(g >= 2)
    def _():
      for src, dst in out_descs(g - 2):
        pltpu.make_async_copy(src, dst, sem_out).wait()

    def cbody(gi, _):
      r = gi >> 3
      sl = pl.ds((gi & 7) * 16, 16)
      tf16 = tf_v[m, r, sl]
      dl16 = dl_v[m, r, sl]
      idf = df_v[r, sl]
      denom = tf16 + K1 * (1.0 - BB + BB * dl16 * (1.0 / AVG_DOC_LEN))
      w_v[m, r, sl] = idf * tf16 * (K1 + 1.0) / denom
      v = idx_v[m, r, sl]
      gidx_v[m, r, sl] = ((v & -8192) + ((v & 4095) << 1) +
                          ((v & 8191) >> 12))
      return 0
    lax.fori_loop(0, ABLK // 16, cbody, 0)

    for src, dst in out_descs(g):
      pltpu.async_copy(src, dst, sem_out)
    return 0

  lax.fori_loop(0, NABLK, block, 0)
  for g in (NABLK - 2, NABLK - 1):
    for src, dst in out_descs(g):
      pltpu.make_async_copy(src, dst, sem_out).wait()


_sc_kernel_a = functools.partial(
    pl.kernel,
    out_type=(jax.ShapeDtypeStruct((ROWS2D, CHUNK), jnp.float32),
              jax.ShapeDtypeStruct((ROWS2D, CHUNK), jnp.int32)),
    mesh=plsc.VectorSubcoreMesh(core_axis_name="c", subcore_axis_name="s"),
    compiler_params=pltpu.CompilerParams(use_tc_tiling_on_sc=False),
    scratch_types=[
        pltpu.VMEM((2, ANCH, CHUNK), jnp.int32),
        pltpu.VMEM((2, ANCH, CHUNK), jnp.float32),
        pltpu.VMEM((2, ANCH, CHUNK), jnp.float32),
        pltpu.VMEM((ANCH, CHUNK), jnp.float32),
        pltpu.VMEM((2, ANCH, CHUNK), jnp.float32),
        pltpu.VMEM((2, ANCH, CHUNK), jnp.int32),
        pltpu.SemaphoreType.DMA,
        pltpu.SemaphoreType.DMA,
        pltpu.SemaphoreType.DMA,
    ],
)(_sc_a_body)


def _sc_b_body(gidx_hbm, bmap_hbm, w_hbm, emb_hbm, out_hbm,
               gidx_v, bmap_v, w_v, rows_v, stage_v, acc_sh,
               sem_meta, sem_gather, sem_scat):
  c = lax.axis_index("c")
  s = lax.axis_index("s")
  wid = s * NC + c
  row_base = wid * ROWS_PER_W

  def meta_descs(g):
    m = g % 3
    sl = pl.ds(row_base + g * NCH, NCH)
    return [
        (gidx_hbm.at[sl], gidx_v.at[m]),
        (bmap_hbm.at[sl], bmap_v.at[m]),
        (w_hbm.at[sl], w_v.at[m]),
    ]

  def issue_meta(g):
    for src, dst in meta_descs(g):
      pltpu.async_copy(src, dst, sem_meta)

  def wait_meta(g):
    for src, dst in meta_descs(g):
      pltpu.make_async_copy(src, dst, sem_meta).wait()

  def gather_descs(g):
    m = g % 3
    p = g % 2
    return [(emb_hbm.at[gidx_v.at[m].at[k]],
             rows_v.at[p].at[pl.ds(k * CHUNK, CHUNK)]) for k in range(NCH)]

  def issue_gather(g):
    for src, dst in gather_descs(g):
      pltpu.async_copy(src, dst, sem_gather)

  def wait_gather(g):
    for src, dst in gather_descs(g):
      pltpu.make_async_copy(src, dst, sem_gather).wait()

  def scat_descs(g):
    m = g % 3
    p = g % 2
    sem = sem_scat.at[p]
    return [(rows_v.at[p].at[pl.ds(k * CHUNK, CHUNK)],
             acc_sh.at[bmap_v.at[m].at[k]], sem) for k in range(NCH)]

  def issue_scat(g):
    for src, dst, sem in scat_descs(g):
      pltpu.async_copy(src, dst, sem, add=True)

  def wait_scat(g):
    for src, dst, sem in scat_descs(g):
      pltpu.make_async_copy(src, dst, sem).wait()

  def zbody(i, _):
    for cc in range(EMBED // 16):
      stage_v[i, pl.ds(cc * 16, 16)] = jnp.zeros((16,), jnp.float32)
    return 0
  lax.fori_loop(0, DOCS_PER_S, zbody, 0)
  pltpu.sync_copy(stage_v, acc_sh.at[pl.ds(s * DOCS_PER_S, DOCS_PER_S)])
  plsc.subcore_barrier()

  issue_meta(0)
  issue_meta(1)
  wait_meta(0)
  issue_gather(0)

  def block(g, _):
    p = g % 2
    m = g % 3
    wait_gather(g)

    @pl.when(g + 1 < NBLK)
    def _():
      wait_meta(g + 1)

    @pl.when(g >= 1)
    def _():
      wait_scat(g - 1)

    @pl.when(g + 1 < NBLK)
    def _():
      issue_gather(g + 1)

    def cbody(gi, _):
      r = gi >> 3
      sl = pl.ds((gi & 7) * 16, 16)
      w16 = w_v[m, r, sl]
      row0 = gi * 16
      for j in range(16):
        wj = _bcast_lane(w16, j)
        for e4 in range(EMBED // 16):
          sl2 = pl.ds(e4 * 16, 16)
          rows_v[p, row0 + j, sl2] = rows_v[p, row0 + j, sl2] * wj
      return 0
    lax.fori_loop(0, BLK // 16, cbody, 0)

    issue_scat(g)

    @pl.when(g + 2 < NBLK)
    def _():
      issue_meta(g + 2)
    return 0

  lax.fori_loop(0, NBLK, block, 0)

  wait_scat(NBLK - 1)
  plsc.subcore_barrier()
  pltpu.sync_copy(acc_sh.at[pl.ds(s * DOCS_PER_S, DOCS_PER_S)], stage_v)
  pltpu.sync_copy(stage_v,
                  out_hbm.at[pl.ds(c * B + s * DOCS_PER_S, DOCS_PER_S)])


_sc_kernel_b = functools.partial(
    pl.kernel,
    out_type=jax.ShapeDtypeStruct((NC * B, EMBED), jnp.float32),
    mesh=plsc.VectorSubcoreMesh(core_axis_name="c", subcore_axis_name="s"),
    compiler_params=pltpu.CompilerParams(use_tc_tiling_on_sc=False),
    scratch_types=[
        pltpu.VMEM((3, NCH, CHUNK), jnp.int32),
        pltpu.VMEM((3, NCH, CHUNK), jnp.int32),
        pltpu.VMEM((3, NCH, CHUNK), jnp.float32),
        pltpu.VMEM((2, BLK, EMBED), jnp.float32),
        pltpu.VMEM((DOCS_PER_S, EMBED), jnp.float32),
        pltpu.VMEM_SHARED((B, EMBED), jnp.float32),
        pltpu.SemaphoreType.DMA,
        pltpu.SemaphoreType.DMA,
        pltpu.SemaphoreType.DMA((2,)),
    ],
)(_sc_b_body)


def _mm_body(acc_ref, w_ref, b_ref, out_ref):
  a = acc_ref[0:B, :] + acc_ref[B:2 * B, :]
  out_ref[...] = (
      jnp.dot(a, w_ref[...], preferred_element_type=jnp.float32) + b_ref[...])


def kernel(all_indices, all_tf, all_doc_len, batch_map, df, emb_table, W, b):
  idx2 = all_indices.astype(jnp.int32).reshape(ROWS2D, CHUNK)
  bm2 = batch_map.astype(jnp.int32).reshape(ROWS2D, CHUNK)
  tf2 = all_tf.reshape(ROWS2D, CHUNK)
  dl2 = all_doc_len.reshape(ROWS2D, CHUNK)
  df_pad = jnp.pad(df, (0, VPAD - VOCAB)).reshape(VPAD // 128, 128)
  idf = pl.pallas_call(
      _idf_body,
      out_shape=jax.ShapeDtypeStruct((VPAD // 128, 128), jnp.float32),
  )(df_pad).reshape(VPAD)
  w2, gidx2 = _sc_kernel_a(idx2, tf2, dl2, idf)
  ngrid = (VOCAB + TCHUNK - 1) // TCHUNK
  emb_lin = pl.pallas_call(
      _tr_body,
      compiler_params=pltpu.CompilerParams(fuse_transposed_lhs_in_matmul=True),
      grid=(ngrid,),
      in_specs=[pl.BlockSpec((EMBED, TCHUNK), lambda i: (0, i))],
      out_specs=pl.BlockSpec((TCHUNK // 2, 2 * EMBED), lambda i: (i, 0)),
      out_shape=jax.ShapeDtypeStruct((ngrid * TCHUNK // 2, 2 * EMBED),
                                     jnp.float32),
  )(emb_table.T).reshape(ngrid * TCHUNK, EMBED)
  acc = _sc_kernel_b(gidx2, bm2, w2, emb_lin)
  logits = pl.pallas_call(
      _mm_body,
      out_shape=jax.ShapeDtypeStruct((B, NUM_CLASSES), jnp.float32),
  )(acc, W, b.reshape(1, NUM_CLASSES))
  return logits


# speedup vs baseline: 1.4389x; 1.4389x over previous
import functools

import jax
import jax.numpy as jnp
from jax import lax
from jax.experimental import pallas as pl
from jax.experimental.pallas import tpu as pltpu
from jax.experimental.pallas import tpu_sc as plsc

VOCAB = 1000000
EMBED = 64
NUM_CLASSES = 128
B = 4096
T = 819200
AVG_DOC_LEN = 200.0
NUM_DOCS = 1000000
K1 = 1.2
BB = 0.75

NC = 2
NS = 16
NW = NC * NS
CHUNK = 128
BLK = 512
NCH = BLK // CHUNK
TOK_PER_W = T // NW
NBLK = TOK_PER_W // BLK
ROWS2D = T // CHUNK
ROWS_PER_W = ROWS2D // NW
DOCS_PER_S = B // NS

VPAD = 1048576


def _idf_body(df_ref, out_ref):
  x = df_ref[...]
  out_ref[...] = jnp.log((NUM_DOCS - x + 0.5) / (x + 0.5) + 1.0)


TCHUNK = 8192


def _tr_body(in_ref, out_ref):
  x = in_ref[...]
  out_ref[:, 0:EMBED] = x[:, 0:TCHUNK // 2].T
  out_ref[:, EMBED:2 * EMBED] = x[:, TCHUNK // 2:TCHUNK].T


_GATHER_DNUMS = lax.GatherDimensionNumbers(
    offset_dims=(), collapsed_slice_dims=(0,), start_index_map=(0,))


def _bcast_lane(v, j):
  idx = jnp.full((16, 1), j, dtype=jnp.int32)
  return lax.gather(v, idx, _GATHER_DNUMS, (1,),
                    mode=lax.GatherScatterMode.PROMISE_IN_BOUNDS)


def _sc_kernel_body(idx_hbm, bmap_hbm, tf_hbm, dl_hbm, df_hbm, emb_hbm,
                    out_hbm, idx_v, bmap_v, tf_v, dl_v, df_v, w_v, gidx_v,
                    rows_v, stage_v, acc_sh, sem_meta, sem_gather, sem_scat):
  c = lax.axis_index("c")
  s = lax.axis_index("s")
  wid = s * NC + c
  row_base = wid * ROWS_PER_W

  def meta_descs(g):
    m = g % 3
    row0 = row_base + g * NCH
    sl = pl.ds(row0, NCH)
    return [
        (idx_hbm.at[sl], idx_v.at[m]),
        (bmap_hbm.at[sl], bmap_v.at[m]),
        (tf_hbm.at[sl], tf_v.at[m]),
        (dl_hbm.at[sl], dl_v.at[m]),
    ]

  def issue_meta(g):
    for src, dst in meta_descs(g):
      pltpu.async_copy(src, dst, sem_meta)

  def wait_meta(g):
    for src, dst in meta_descs(g):
      pltpu.make_async_copy(src, dst, sem_meta).wait()

  def fill_gidx(g):
    m = g % 3

    def dbody(i, _):
      r = i >> 3
      sl = pl.ds((i & 7) * 16, 16)
      v = idx_v[m, r, sl]
      gidx_v[r, sl] = ((v & -8192) + ((v & 4095) << 1) + ((v & 8191) >> 12))
      return 0
    lax.fori_loop(0, NCH * 8, dbody, 0)

  def gather_descs(g):
    m = g % 3
    p = g % 2
    out = []
    for k in range(NCH):
      out.append((emb_hbm.at[gidx_v.at[k]],
                  rows_v.at[p].at[pl.ds(k * CHUNK, CHUNK)]))
      out.append((df_hbm.at[idx_v.at[m].at[k]], df_v.at[p].at[k]))
    return out

  def issue_gather(g):
    for src, dst in gather_descs(g):
      pltpu.async_copy(src, dst, sem_gather)

  def wait_gather(g):
    for src, dst in gather_descs(g):
      pltpu.make_async_copy(src, dst, sem_gather).wait()

  def scat_descs(g):
    m = g % 3
    p = g % 2
    sem = sem_scat.at[p]
    out = []
    for k in range(NCH):
      out.append((rows_v.at[p].at[pl.ds(k * CHUNK, CHUNK)],
                  acc_sh.at[bmap_v.at[m].at[k]], sem))
    return out

  def issue_scat(g):
    for src, dst, sem in scat_descs(g):
      pltpu.async_copy(src, dst, sem, add=True)

  def wait_scat(g):
    for src, dst, sem in scat_descs(g):
      pltpu.make_async_copy(src, dst, sem).wait()

  def zbody(i, _):
    for cc in range(EMBED // 16):
      stage_v[i, pl.ds(cc * 16, 16)] = jnp.zeros((16,), jnp.float32)
    return 0
  lax.fori_loop(0, DOCS_PER_S, zbody, 0)
  pltpu.sync_copy(stage_v, acc_sh.at[pl.ds(s * DOCS_PER_S, DOCS_PER_S)])
  plsc.subcore_barrier()

  issue_meta(0)
  issue_meta(1)
  wait_meta(0)
  fill_gidx(0)
  issue_gather(0)

  def block(g, _):
    p = g % 2
    m = g % 3
    wait_gather(g)

    @pl.when(g + 1 < NBLK)
    def _():
      wait_meta(g + 1)

    @pl.when(g >= 1)
    def _():
      wait_scat(g - 1)

    @pl.when(g + 1 < NBLK)
    def _():
      fill_gidx(g + 1)
      issue_gather(g + 1)

    def cbody(gi, _):
      r = gi >> 3
      cc = (gi & 7) * 16
      sl = pl.ds(cc, 16)
      tf16 = tf_v[m, r, sl]
      dl16 = dl_v[m, r, sl]
      idf = df_v[p, r, sl]
      denom = tf16 + K1 * (1.0 - BB + BB * dl16 * (1.0 / AVG_DOC_LEN))
      w16 = idf * tf16 * (K1 + 1.0) / denom
      row0 = gi * 16
      for j in range(16):
        wj = _bcast_lane(w16, j)
        for e4 in range(EMBED // 16):
          sl2 = pl.ds(e4 * 16, 16)
          rows_v[p, row0 + j, sl2] = rows_v[p, row0 + j, sl2] * wj
      return 0
    lax.fori_loop(0, BLK // 16, cbody, 0)

    issue_scat(g)

    @pl.when(g + 2 < NBLK)
    def _():
      issue_meta(g + 2)
    return 0

  lax.fori_loop(0, NBLK, block, 0)

  wait_scat(NBLK - 1)
  plsc.subcore_barrier()
  pltpu.sync_copy(acc_sh.at[pl.ds(s * DOCS_PER_S, DOCS_PER_S)], stage_v)
  pltpu.sync_copy(stage_v,
                  out_hbm.at[pl.ds(c * B + s * DOCS_PER_S, DOCS_PER_S)])


_sc_kernel = functools.partial(
    pl.kernel,
    out_type=jax.ShapeDtypeStruct((NC * B, EMBED), jnp.float32),
    mesh=plsc.VectorSubcoreMesh(core_axis_name="c", subcore_axis_name="s"),
    compiler_params=pltpu.CompilerParams(use_tc_tiling_on_sc=False),
    scratch_types=[
        pltpu.VMEM((3, NCH, CHUNK), jnp.int32),
        pltpu.VMEM((3, NCH, CHUNK), jnp.int32),
        pltpu.VMEM((3, NCH, CHUNK), jnp.float32),
        pltpu.VMEM((3, NCH, CHUNK), jnp.float32),
        pltpu.VMEM((2, NCH, CHUNK), jnp.float32),
        pltpu.VMEM((NCH, CHUNK), jnp.float32),
        pltpu.VMEM((NCH, CHUNK), jnp.int32),
        pltpu.VMEM((2, BLK, EMBED), jnp.float32),
        pltpu.VMEM((DOCS_PER_S, EMBED), jnp.float32),
        pltpu.VMEM_SHARED((B, EMBED), jnp.float32),
        pltpu.SemaphoreType.DMA,
        pltpu.SemaphoreType.DMA,
        pltpu.SemaphoreType.DMA((2,)),
    ],
)(_sc_kernel_body)


def _mm_body(acc_ref, w_ref, b_ref, out_ref):
  a = acc_ref[0:B, :] + acc_ref[B:2 * B, :]
  out_ref[...] = (
      jnp.dot(a, w_ref[...], preferred_element_type=jnp.float32) + b_ref[...])


def kernel(all_indices, all_tf, all_doc_len, batch_map, df, emb_table, W, b):
  idx2 = all_indices.astype(jnp.int32).reshape(ROWS2D, CHUNK)
  bm2 = batch_map.astype(jnp.int32).reshape(ROWS2D, CHUNK)
  tf2 = all_tf.reshape(ROWS2D, CHUNK)
  dl2 = all_doc_len.reshape(ROWS2D, CHUNK)
  df_pad = jnp.pad(df, (0, VPAD - VOCAB)).reshape(VPAD // 128, 128)
  idf = pl.pallas_call(
      _idf_body,
      out_shape=jax.ShapeDtypeStruct((VPAD // 128, 128), jnp.float32),
  )(df_pad).reshape(VPAD)
  ngrid = (VOCAB + TCHUNK - 1) // TCHUNK
  emb_lin = pl.pallas_call(
      _tr_body,
      grid=(ngrid,),
      in_specs=[pl.BlockSpec((EMBED, TCHUNK), lambda i: (0, i))],
      out_specs=pl.BlockSpec((TCHUNK // 2, 2 * EMBED), lambda i: (i, 0)),
      out_shape=jax.ShapeDtypeStruct((ngrid * TCHUNK // 2, 2 * EMBED),
                                     jnp.float32),
  )(emb_table.T).reshape(ngrid * TCHUNK, EMBED)
  acc = _sc_kernel(idx2, bm2, tf2, dl2, idf, emb_lin)
  logits = pl.pallas_call(
      _mm_body,
      out_shape=jax.ShapeDtypeStruct((B, NUM_CLASSES), jnp.float32),
  )(acc, W, b.reshape(1, NUM_CLASSES))
  return logits
